# baseline (speedup 1.0000x reference)
import functools

import jax
import jax.numpy as jnp
from jax import lax
from jax.experimental import pallas as pl
from jax.experimental.pallas import tpu as pltpu
from jax.experimental.pallas import tpu_sc as plsc

B = 16384
D = 64
L = 16
NC = 2
NS = 16
NW = NC * NS
BPW = B // NW
NG = BPW // L


def _mf_body(uid_hbm, mid_hbm, uemb_hbm, memb_hbm, ubias_hbm, mbias_hbm,
             out_hbm,
             uid_v, mid_v, urows_v, mrows_v, ub_v, mb_v, out_v, sem):
    wid = lax.axis_index("s") * NC + lax.axis_index("c")
    base = wid * BPW

    pltpu.sync_copy(uid_hbm.at[pl.ds(base, BPW)], uid_v)
    pltpu.sync_copy(mid_hbm.at[pl.ds(base, BPW)], mid_v)

    cps = [
        pltpu.async_copy(uemb_hbm.at[uid_v], urows_v, sem),
        pltpu.async_copy(memb_hbm.at[mid_v], mrows_v, sem),
        pltpu.async_copy(ubias_hbm.at[uid_v], ub_v, sem),
        pltpu.async_copy(mbias_hbm.at[mid_v], mb_v, sem),
    ]
    for cp in cps:
        cp.wait()

    lanes = lax.iota(jnp.int32, L)

    def group_body(g, carry):
        r0 = pl.multiple_of(g * L, L)
        dots = jnp.zeros((L,), jnp.float32)
        for j in range(L):
            r = r0 + j
            p0 = urows_v[r, pl.ds(0, L)] * mrows_v[r, pl.ds(0, L)]
            p1 = urows_v[r, pl.ds(L, L)] * mrows_v[r, pl.ds(L, L)]
            p2 = urows_v[r, pl.ds(2 * L, L)] * mrows_v[r, pl.ds(2 * L, L)]
            p3 = urows_v[r, pl.ds(3 * L, L)] * mrows_v[r, pl.ds(3 * L, L)]
            s = jnp.sum((p0 + p1) + (p2 + p3))
            dots = jnp.where(lanes == j, s, dots)
        sl = pl.ds(r0, L)
        out_v[sl] = dots + ub_v[sl] + mb_v[sl]
        return carry

    lax.fori_loop(0, NG, group_body, 0)

    pltpu.sync_copy(out_v, out_hbm.at[pl.ds(base, BPW)])


@jax.jit
def _mf_call(user_ids, movie_ids, user_emb, movie_emb, user_bias, movie_bias):
    mesh = plsc.VectorSubcoreMesh(core_axis_name="c", subcore_axis_name="s")
    run = pl.kernel(
        _mf_body,
        mesh=mesh,
        compiler_params=pltpu.CompilerParams(
            needs_layout_passes=False, use_tc_tiling_on_sc=False
        ),
        out_type=jax.ShapeDtypeStruct((B,), jnp.float32),
        scratch_types=[
            pltpu.VMEM((BPW,), jnp.int32),
            pltpu.VMEM((BPW,), jnp.int32),
            pltpu.VMEM((BPW, D), jnp.float32),
            pltpu.VMEM((BPW, D), jnp.float32),
            pltpu.VMEM((BPW,), jnp.float32),
            pltpu.VMEM((BPW,), jnp.float32),
            pltpu.VMEM((BPW,), jnp.float32),
            pltpu.SemaphoreType.DMA,
        ],
    )
    return run(user_ids, movie_ids, user_emb, movie_emb, user_bias, movie_bias)


def kernel(user_ids, movie_ids, user_emb, movie_emb, user_bias, movie_bias):
    return _mf_call(
        user_ids.astype(jnp.int32),
        movie_ids.astype(jnp.int32),
        user_emb,
        movie_emb,
        user_bias.reshape(-1),
        movie_bias.reshape(-1),
    )
